# Initial kernel scaffold; baseline (speedup 1.0000x reference)
import jax
import jax.numpy as jnp
from jax.experimental import pallas as pl


def kernel(x_main, edge_attr_main, edge_index_main, graph_ids_main, x_bb1, edge_attr_bb1, edge_index_bb1, graph_ids_bb1, x_bb2, edge_attr_bb2, edge_index_bb2, graph_ids_bb2, x_bb3, edge_attr_bb3, edge_index_bb3, graph_ids_bb3, protein_embedding, W_node, W_edge, W_prot, W_bb, W_final, Wl, Wel, Asrc, Adst, Aedge, Woutl):
    raise NotImplementedError("write your pallas kernel here")


# speedup vs baseline: 1.0911x; 1.0911x over previous
import functools

import jax
import jax.numpy as jnp
from jax.experimental import pallas as pl

HID = 64
HEADS = 4
HD = 16
L = 5
B = 1024


def _mm_body(a_ref, b_ref, o_ref):
    o_ref[...] = jnp.dot(a_ref[...], b_ref[...], preferred_element_type=jnp.float32)


def _mm(a, b, blk=1024):
    n, k = a.shape
    m = b.shape[1]
    grid = (pl.cdiv(n, blk),)
    return pl.pallas_call(
        _mm_body,
        grid=grid,
        in_specs=[
            pl.BlockSpec((blk, k), lambda i: (i, 0)),
            pl.BlockSpec((k, m), lambda i: (0, 0)),
        ],
        out_specs=pl.BlockSpec((blk, m), lambda i: (i, 0)),
        out_shape=jax.ShapeDtypeStruct((n, m), jnp.float32),
    )(a, b)


def _run_graph(x, eattr, ei, gids, W_node, W_edge, Wl, Wel, Asrc, Adst, Aedge, Woutl):
    n = x.shape[0]
    h = _mm(x, W_node)
    src, dst = ei[0], ei[1]
    Cw = jnp.einsum('eh,lhkd,lkd->elk', W_edge, Wel.reshape(L, HID, HEADS, HD), Aedge)
    es_all = _mm(eattr, Cw.reshape(6, L * HEADS)).reshape(-1, L, HEADS)
    for i in range(L):
        Wh = _mm(h, Wl[i])
        Whr = Wh.reshape(n, HEADS, HD)
        ssrc = (Whr * Asrc[i]).sum(-1)
        sdst = (Whr * Adst[i]).sum(-1)
        ub = jax.nn.leaky_relu(sdst + ssrc.max() + es_all[:, i].max(), 0.2)
        t = ssrc[src] + sdst[dst] + es_all[:, i]
        sc = jax.nn.leaky_relu(t, 0.2)
        e = jnp.exp(sc - ub[dst])
        d = jax.ops.segment_sum(e, dst, num_segments=n)
        alpha = e / (d[dst] + 1e-16)
        msg = Wh[src].reshape(-1, HEADS, HD) * alpha[..., None]
        agg = jax.ops.segment_sum(msg, dst, num_segments=n).reshape(n, HID)
        h = jax.nn.relu(h + _mm(agg, Woutl[i]))
    return jax.ops.segment_sum(h, gids, num_segments=B)


def kernel(x_main, edge_attr_main, edge_index_main, graph_ids_main,
           x_bb1, edge_attr_bb1, edge_index_bb1, graph_ids_bb1,
           x_bb2, edge_attr_bb2, edge_index_bb2, graph_ids_bb2,
           x_bb3, edge_attr_bb3, edge_index_bb3, graph_ids_bb3,
           protein_embedding, W_node, W_edge, W_prot, W_bb, W_final,
           Wl, Wel, Asrc, Adst, Aedge, Woutl):
    main_out = _run_graph(x_main, edge_attr_main, edge_index_main, graph_ids_main,
                          W_node, W_edge, Wl, Wel, Asrc, Adst, Aedge, Woutl)
    bb_outs = []
    for x, ea, ei, g in ((x_bb1, edge_attr_bb1, edge_index_bb1, graph_ids_bb1),
                         (x_bb2, edge_attr_bb2, edge_index_bb2, graph_ids_bb2),
                         (x_bb3, edge_attr_bb3, edge_index_bb3, graph_ids_bb3)):
        bb = _run_graph(x, ea, ei, g, W_node, W_edge, Wl, Wel, Asrc, Adst, Aedge, Woutl)
        bb_outs.append(_mm(bb, W_bb))
    prot = _mm(protein_embedding, W_prot)
    out = jnp.concatenate([main_out, prot] + bb_outs, axis=1) @ W_final
    return jax.nn.sigmoid(out)


# speedup vs baseline: 44.5473x; 40.8287x over previous
import functools

import jax
import jax.numpy as jnp
from jax import lax
from jax.experimental import pallas as pl
from jax.experimental.pallas import tpu as pltpu
from jax.experimental.pallas import tpu_sc as plsc

HID = 64
HEADS = 4
HD = 16
L = 5
B = 1024

NLANE = 16
NTILE = 32
SUP = 2048
CH = 128
NCH = SUP // CH


def _cdiv(a, b):
    return -(-a // b)


def _mesh():
    return plsc.VectorSubcoreMesh(core_axis_name="c", subcore_axis_name="s",
                                  num_cores=2, num_subcores=16)


def _mm_body(a_ref, b_ref, o_ref):
    o_ref[...] = jnp.dot(a_ref[...], b_ref[...], preferred_element_type=jnp.float32)


def _mm(a, b, blk=1024):
    n, k = a.shape
    m = b.shape[1]
    return pl.pallas_call(
        _mm_body,
        grid=(pl.cdiv(n, blk),),
        in_specs=[
            pl.BlockSpec((blk, k), lambda i: (i, 0)),
            pl.BlockSpec((k, m), lambda i: (0, 0)),
        ],
        out_specs=pl.BlockSpec((blk, m), lambda i: (i, 0)),
        out_shape=jax.ShapeDtypeStruct((n, m), jnp.float32),
    )(a, b)


def _zero_vec(ref, nwords):
    @plsc.parallel_loop(0, nwords, NLANE)
    def _(i):
        ref[pl.ds(i, NLANE)] = jnp.zeros((NLANE,), jnp.float32)


def _chunks(total, cmax=2048):
    out, off = [], 0
    while off < total:
        c = min(cmax, total - off)
        out.append((off, c))
        off += c
    return out


@functools.cache
def _build_k1(nd, nsup_tot):
    nsup_t = nsup_tot // NTILE
    pht = nd // 16

    def body(tabs, srcc, dstc, es, eout, denp,
             tab_v, srcc_v, dstc_v, es_v, e_v, zb_v, den_sh):
        cid = lax.axis_index("c")
        sid = lax.axis_index("s")
        wid = cid * 16 + sid

        _zero_vec(zb_v, 2048)
        for h in range(HEADS):
            for off, c in _chunks(pht):
                pltpu.sync_copy(zb_v.at[pl.ds(0, c)],
                                den_sh.at[pl.ds(sid * pht + off, c)])
            plsc.subcore_barrier()
            pltpu.sync_copy(tabs.at[h], tab_v)

            def sup_body(k, carry, h=h):
                sup = wid * nsup_t + k
                pltpu.sync_copy(srcc.at[sup], srcc_v)
                pltpu.sync_copy(dstc.at[sup], dstc_v)
                pltpu.sync_copy(es.at[h, sup], es_v)

                for c in range(NCH):
                    @plsc.parallel_loop(0, CH, NLANE, unroll=4)
                    def _(j, c=c):
                        sv = srcc_v[c, pl.ds(j, NLANE)]
                        dv = dstc_v[c, pl.ds(j, NLANE)]
                        s1 = plsc.load_gather(tab_v, [sv * 2])
                        s2 = plsc.load_gather(tab_v, [dv * 2 + 1])
                        t = s1 + s2 + es_v[pl.ds(c * CH + j, NLANE)]
                        sc = jnp.maximum(t, 0.2 * t)
                        ub = jnp.maximum(s2, 0.2 * s2)
                        e_v[pl.ds(c * CH + j, NLANE)] = jnp.exp(sc - ub)

                    pltpu.sync_copy(e_v.at[pl.ds(c * CH, CH)],
                                    den_sh.at[dstc_v.at[c]], add=True)
                pltpu.sync_copy(e_v, eout.at[h, sup])
                return carry

            lax.fori_loop(0, nsup_t, sup_body, 0)

            plsc.subcore_barrier()
            for off, c in _chunks(pht):
                sl = pl.ds(sid * pht + off, c)
                pltpu.sync_copy(den_sh.at[sl], denp.at[cid, h].at[sl])
            plsc.subcore_barrier()

    return pl.kernel(
        body,
        out_type=(
            jax.ShapeDtypeStruct((HEADS, nsup_tot, SUP), jnp.float32),
            jax.ShapeDtypeStruct((2, HEADS, nd), jnp.float32),
        ),
        mesh=_mesh(),
        compiler_params=pltpu.CompilerParams(needs_layout_passes=False, use_tc_tiling_on_sc=False),
        scratch_types=[
            pltpu.VMEM((2 * nd,), jnp.float32),
            pltpu.VMEM((NCH, CH), jnp.int32),
            pltpu.VMEM((NCH, CH), jnp.int32),
            pltpu.VMEM((SUP,), jnp.float32),
            pltpu.VMEM((SUP,), jnp.float32),
            pltpu.VMEM((2048,), jnp.float32),
            pltpu.VMEM_SHARED((nd,), jnp.float32),
        ],
    )


@functools.cache
def _build_k2(nd, nsup_tot):
    nsup_t = nsup_tot // NTILE
    prt = nd // 16
    nzc = prt // CH

    def body(whh, srcc, dstc, eh, invd, aggp,
             invd_v, srcc_v, dstc_v, e_v, al_v, wh_v, zb_v, agg_sh, sem):
        cid = lax.axis_index("c")
        sid = lax.axis_index("s")
        wid = cid * 16 + sid

        @plsc.parallel_loop(0, CH, 1)
        def _(r):
            zb_v[r, :] = jnp.zeros((NLANE,), jnp.float32)

        for h in range(HEADS):
            for z in range(nzc):
                pltpu.sync_copy(
                    zb_v, agg_sh.at[pl.ds(sid * prt + z * CH, CH), :])
            plsc.subcore_barrier()

            pltpu.sync_copy(invd.at[h], invd_v)

            def sup_body(k, carry, h=h):
                sup = wid * nsup_t + k
                pltpu.sync_copy(srcc.at[sup], srcc_v)
                pltpu.sync_copy(dstc.at[sup], dstc_v)
                pltpu.sync_copy(eh.at[h, sup], e_v)

                for half in range(2):
                    hc = NCH // 2
                    descs = [
                        pltpu.async_copy(whh.at[h].at[srcc_v.at[half * hc + c]],
                                         wh_v.at[pl.ds(c * CH, CH), :], sem)
                        for c in range(hc)
                    ]
                    for d in descs:
                        d.wait()

                    for c in range(hc):
                        @plsc.parallel_loop(0, CH, NLANE, unroll=2)
                        def _(j, c=c, half=half):
                            cc = half * hc + c
                            dv = dstc_v[cc, pl.ds(j, NLANE)]
                            iv = plsc.load_gather(invd_v, [dv])
                            al_v[pl.ds(c * CH + j, NLANE)] = (
                                e_v[pl.ds(cc * CH + j, NLANE)] * iv)

                    @plsc.parallel_loop(0, SUP // 2, NLANE)
                    def _(i):
                        av = al_v[pl.ds(i, NLANE)]
                        for j in range(NLANE):
                            wh_v[i + j, :] = wh_v[i + j, :] * av[j]

                    for c in range(hc):
                        pltpu.sync_copy(wh_v.at[pl.ds(c * CH, CH), :],
                                        agg_sh.at[dstc_v.at[half * hc + c]],
                                        add=True)
                return carry

            lax.fori_loop(0, nsup_t, sup_body, 0)

            plsc.subcore_barrier()
            for z in range(nzc):
                sl = pl.ds(sid * prt + z * CH, CH)
                pltpu.sync_copy(agg_sh.at[sl, :], aggp.at[cid, h].at[sl, :])
            plsc.subcore_barrier()

    return pl.kernel(
        body,
        out_type=jax.ShapeDtypeStruct((2, HEADS, nd, HD), jnp.float32),
        mesh=_mesh(),
        compiler_params=pltpu.CompilerParams(needs_layout_passes=False, use_tc_tiling_on_sc=False),
        scratch_types=[
            pltpu.VMEM((nd,), jnp.float32),
            pltpu.VMEM((NCH, CH), jnp.int32),
            pltpu.VMEM((NCH, CH), jnp.int32),
            pltpu.VMEM((SUP,), jnp.float32),
            pltpu.VMEM((SUP // 2,), jnp.float32),
            pltpu.VMEM((SUP // 2, HD), jnp.float32),
            pltpu.VMEM((CH, HD), jnp.float32),
            pltpu.VMEM_SHARED((nd, HD), jnp.float32),
            pltpu.SemaphoreType.DMA,
        ],
    )


BD = 1088


@functools.cache
def _build_k3(nr):
    nrt = nr // (NTILE * CH)
    prt = BD // 16
    rchunks = _chunks(prt, 32)

    def body(hp, gidc, outp, gidc_v, h_v, zb_v, out_sh):
        cid = lax.axis_index("c")
        sid = lax.axis_index("s")
        wid = cid * 16 + sid

        @plsc.parallel_loop(0, 32, 1)
        def _(r):
            for j in range(4):
                zb_v[r, pl.ds(j * NLANE, NLANE)] = jnp.zeros((NLANE,), jnp.float32)

        for off, rc in rchunks:
            pltpu.sync_copy(zb_v.at[pl.ds(0, rc), :],
                            out_sh.at[pl.ds(sid * prt + off, rc), :])
        plsc.subcore_barrier()

        pltpu.sync_copy(gidc.at[pl.ds(wid * nrt, nrt)], gidc_v)
        for c in range(nrt):
            pltpu.sync_copy(hp.at[pl.ds((wid * nrt + c) * CH, CH), :], h_v)
            pltpu.sync_copy(h_v, out_sh.at[gidc_v.at[c]], add=True)

        plsc.subcore_barrier()
        for off, rc in rchunks:
            sl = pl.ds(sid * prt + off, rc)
            pltpu.sync_copy(out_sh.at[sl, :], outp.at[cid].at[sl, :])

    return pl.kernel(
        body,
        out_type=jax.ShapeDtypeStruct((2, BD, HID), jnp.float32),
        mesh=_mesh(),
        compiler_params=pltpu.CompilerParams(needs_layout_passes=False, use_tc_tiling_on_sc=False),
        scratch_types=[
            pltpu.VMEM((nrt, CH), jnp.int32),
            pltpu.VMEM((CH, HID), jnp.float32),
            pltpu.VMEM((32, HID), jnp.float32),
            pltpu.VMEM_SHARED((BD, HID), jnp.float32),
        ],
    )


def _run_graph(x, eattr, ei, gids, W_node, W_edge, Wl, Wel, Asrc, Adst, Aedge, Woutl):
    n = x.shape[0]
    ne = ei.shape[1]
    nd = _cdiv(n + 1, SUP) * SUP
    nsup_tot = _cdiv(ne, NTILE * SUP) * NTILE
    ne_pad = nsup_tot * SUP
    nr = _cdiv(n, NTILE * CH) * NTILE * CH

    h = _mm(x, W_node)
    src = jnp.concatenate([ei[0], jnp.full((ne_pad - ne,), n, jnp.int32)])
    dst = jnp.concatenate([ei[1], jnp.full((ne_pad - ne,), n, jnp.int32)])
    srcc = src.reshape(nsup_tot, NCH, CH)
    dstc = dst.reshape(nsup_tot, NCH, CH)

    Cw = jnp.einsum('eh,lhkd,lkd->elk', W_edge, Wel.reshape(L, HID, HEADS, HD), Aedge)
    es_all = _mm(eattr, Cw.reshape(6, L * HEADS)).reshape(ne, L, HEADS)

    k1 = _build_k1(nd, nsup_tot)
    k2 = _build_k2(nd, nsup_tot)

    for i in range(L):
        Wh = _mm(h, Wl[i])
        Whr = Wh.reshape(n, HEADS, HD)
        ssrc = (Whr * Asrc[i]).sum(-1)
        sdst = (Whr * Adst[i]).sum(-1)
        es_l = es_all[:, i, :]
        Kh = ssrc.max(0) + es_l.max(0)

        tab = jnp.stack([ssrc, sdst + Kh[None, :]], axis=-1)
        tab = jnp.transpose(tab, (1, 0, 2)).reshape(HEADS, 2 * n)
        tabs = jnp.pad(tab, ((0, 0), (0, 2 * nd - 2 * n)))

        esh = jnp.transpose(es_l - Kh[None, :], (1, 0))
        esh = jnp.pad(esh, ((0, 0), (0, ne_pad - ne))).reshape(HEADS, nsup_tot, SUP)

        whh = jnp.transpose(Whr, (1, 0, 2))
        whh = jnp.pad(whh, ((0, 0), (0, nd - n), (0, 0)))

        eout, denp = k1(tabs, srcc, dstc, esh)
        den = denp[0] + denp[1]
        invd = 1.0 / (den + 1e-16)

        aggp = k2(whh, srcc, dstc, eout, invd)
        agg = aggp[0] + aggp[1]
        agg = jnp.transpose(agg[:, :n, :], (1, 0, 2)).reshape(n, HID)
        h = jax.nn.relu(h + _mm(agg, Woutl[i]))

    hp = jnp.pad(h, ((0, nr - n), (0, 0)))
    gid = jnp.concatenate([gids.astype(jnp.int32),
                           jnp.full((nr - n,), B, jnp.int32)])
    gidc = gid.reshape(nr // CH, CH)
    outp = _build_k3(nr)(hp, gidc)
    return (outp[0] + outp[1])[:B]


def kernel(x_main, edge_attr_main, edge_index_main, graph_ids_main,
           x_bb1, edge_attr_bb1, edge_index_bb1, graph_ids_bb1,
           x_bb2, edge_attr_bb2, edge_index_bb2, graph_ids_bb2,
           x_bb3, edge_attr_bb3, edge_index_bb3, graph_ids_bb3,
           protein_embedding, W_node, W_edge, W_prot, W_bb, W_final,
           Wl, Wel, Asrc, Adst, Aedge, Woutl):
    main_out = _run_graph(x_main, edge_attr_main, edge_index_main, graph_ids_main,
                          W_node, W_edge, Wl, Wel, Asrc, Adst, Aedge, Woutl)
    bb_outs = []
    for x, ea, ei, g in ((x_bb1, edge_attr_bb1, edge_index_bb1, graph_ids_bb1),
                         (x_bb2, edge_attr_bb2, edge_index_bb2, graph_ids_bb2),
                         (x_bb3, edge_attr_bb3, edge_index_bb3, graph_ids_bb3)):
        bb = _run_graph(x, ea, ei, g, W_node, W_edge, Wl, Wel, Asrc, Adst, Aedge, Woutl)
        bb_outs.append(_mm(bb, W_bb))
    prot = _mm(protein_embedding, W_prot)
    out = jnp.concatenate([main_out, prot] + bb_outs, axis=1) @ W_final
    return jax.nn.sigmoid(out)


# speedup vs baseline: 51.0899x; 1.1469x over previous
import functools

import jax
import jax.numpy as jnp
from jax import lax
from jax.experimental import pallas as pl
from jax.experimental.pallas import tpu as pltpu
from jax.experimental.pallas import tpu_sc as plsc

HID = 64
HEADS = 4
HD = 16
L = 5
B = 1024

NLANE = 16
NTILE = 32
SUP = 2048
CH = 128
NCH = SUP // CH


def _cdiv(a, b):
    return -(-a // b)


def _mesh():
    return plsc.VectorSubcoreMesh(core_axis_name="c", subcore_axis_name="s",
                                  num_cores=2, num_subcores=16)


def _mm_body(a_ref, b_ref, o_ref):
    o_ref[...] = jnp.dot(a_ref[...], b_ref[...], preferred_element_type=jnp.float32)


def _mm(a, b, blk=1024):
    n, k = a.shape
    m = b.shape[1]
    return pl.pallas_call(
        _mm_body,
        grid=(pl.cdiv(n, blk),),
        in_specs=[
            pl.BlockSpec((blk, k), lambda i: (i, 0)),
            pl.BlockSpec((k, m), lambda i: (0, 0)),
        ],
        out_specs=pl.BlockSpec((blk, m), lambda i: (i, 0)),
        out_shape=jax.ShapeDtypeStruct((n, m), jnp.float32),
    )(a, b)


def _zero_vec(ref, nwords):
    @plsc.parallel_loop(0, nwords, NLANE)
    def _(i):
        ref[pl.ds(i, NLANE)] = jnp.zeros((NLANE,), jnp.float32)


def _chunks(total, cmax=2048):
    out, off = [], 0
    while off < total:
        c = min(cmax, total - off)
        out.append((off, c))
        off += c
    return out


@functools.cache
def _build_k1(nd, nsup_tot):
    nsup_t = nsup_tot // NTILE
    pht = nd // 16

    def body(tabs, srcc, dstc, es, eout, denp,
             tab_v, srcc_v, dstc_v, es_v, e_v, zb_v, den_sh, sem, sem2):
        cid = lax.axis_index("c")
        sid = lax.axis_index("s")
        wid = cid * 16 + sid

        _zero_vec(zb_v, 2048)
        for h in range(HEADS):
            for off, c in _chunks(pht):
                pltpu.sync_copy(zb_v.at[pl.ds(0, c)],
                                den_sh.at[pl.ds(sid * pht + off, c)])
            plsc.subcore_barrier()
            pltpu.sync_copy(tabs.at[h], tab_v)

            def sup_body(k, carry, h=h):
                sup = wid * nsup_t + k
                loads = [pltpu.async_copy(srcc.at[sup], srcc_v, sem),
                         pltpu.async_copy(dstc.at[sup], dstc_v, sem),
                         pltpu.async_copy(es.at[h, sup], es_v, sem)]
                for d in loads:
                    d.wait()

                scats = []
                for c in range(NCH):
                    @plsc.parallel_loop(0, CH, NLANE, unroll=4)
                    def _(j, c=c):
                        sv = srcc_v[c, pl.ds(j, NLANE)]
                        dv = dstc_v[c, pl.ds(j, NLANE)]
                        s1 = plsc.load_gather(tab_v, [sv * 2])
                        s2 = plsc.load_gather(tab_v, [dv * 2 + 1])
                        t = s1 + s2 + es_v[pl.ds(c * CH + j, NLANE)]
                        sc = jnp.maximum(t, 0.2 * t)
                        ub = jnp.maximum(s2, 0.2 * s2)
                        e_v[pl.ds(c * CH + j, NLANE)] = jnp.exp(sc - ub)

                    scats.append(
                        pltpu.async_copy(e_v.at[pl.ds(c * CH, CH)],
                                         den_sh.at[dstc_v.at[c]], sem2,
                                         add=True))
                for d in scats:
                    d.wait()
                pltpu.sync_copy(e_v, eout.at[h, sup])
                return carry

            lax.fori_loop(0, nsup_t, sup_body, 0)

            plsc.subcore_barrier()
            for off, c in _chunks(pht):
                sl = pl.ds(sid * pht + off, c)
                pltpu.sync_copy(den_sh.at[sl], denp.at[cid, h].at[sl])
            plsc.subcore_barrier()

    return pl.kernel(
        body,
        out_type=(
            jax.ShapeDtypeStruct((HEADS, nsup_tot, SUP), jnp.float32),
            jax.ShapeDtypeStruct((2, HEADS, nd), jnp.float32),
        ),
        mesh=_mesh(),
        compiler_params=pltpu.CompilerParams(needs_layout_passes=False, use_tc_tiling_on_sc=False),
        scratch_types=[
            pltpu.VMEM((2 * nd,), jnp.float32),
            pltpu.VMEM((NCH, CH), jnp.int32),
            pltpu.VMEM((NCH, CH), jnp.int32),
            pltpu.VMEM((SUP,), jnp.float32),
            pltpu.VMEM((SUP,), jnp.float32),
            pltpu.VMEM((2048,), jnp.float32),
            pltpu.VMEM_SHARED((nd,), jnp.float32),
            pltpu.SemaphoreType.DMA,
            pltpu.SemaphoreType.DMA,
        ],
    )


@functools.cache
def _build_k2(nd, nsup_tot):
    nsup_t = nsup_tot // NTILE
    prt = nd // 16
    nzc = prt // CH

    def body(whh, srcc, dstc, eh, aggp,
             srcc_v, dstc_v, e_v, wh_v, zb_v, agg_sh, sem, sem2):
        cid = lax.axis_index("c")
        sid = lax.axis_index("s")
        wid = cid * 16 + sid

        @plsc.parallel_loop(0, CH, 1)
        def _(r):
            zb_v[r, :] = jnp.zeros((NLANE,), jnp.float32)

        for h in range(HEADS):
            for z in range(nzc):
                pltpu.sync_copy(
                    zb_v, agg_sh.at[pl.ds(sid * prt + z * CH, CH), :])
            plsc.subcore_barrier()

            def sup_body(k, carry, h=h):
                sup = wid * nsup_t + k
                loads = [pltpu.async_copy(srcc.at[sup], srcc_v, sem2),
                         pltpu.async_copy(dstc.at[sup], dstc_v, sem2),
                         pltpu.async_copy(eh.at[h, sup], e_v, sem2)]
                for d in loads:
                    d.wait()

                gathers = [
                    pltpu.async_copy(whh.at[h].at[srcc_v.at[c]],
                                     wh_v.at[pl.ds(c * CH, CH), :], sem)
                    for c in range(NCH)
                ]
                scats = []
                for c in range(NCH):
                    gathers[c].wait()

                    @plsc.parallel_loop(0, CH, NLANE)
                    def _(i, c=c):
                        av = e_v[pl.ds(c * CH + i, NLANE)]
                        for j in range(NLANE):
                            wh_v[c * CH + i + j, :] = (
                                wh_v[c * CH + i + j, :] * av[j])

                    scats.append(
                        pltpu.async_copy(wh_v.at[pl.ds(c * CH, CH), :],
                                         agg_sh.at[dstc_v.at[c]], sem2,
                                         add=True))
                for d in scats:
                    d.wait()
                return carry

            lax.fori_loop(0, nsup_t, sup_body, 0)

            plsc.subcore_barrier()
            for z in range(nzc):
                sl = pl.ds(sid * prt + z * CH, CH)
                pltpu.sync_copy(agg_sh.at[sl, :], aggp.at[cid, h].at[sl, :])
            plsc.subcore_barrier()

    return pl.kernel(
        body,
        out_type=jax.ShapeDtypeStruct((2, HEADS, nd, HD), jnp.float32),
        mesh=_mesh(),
        compiler_params=pltpu.CompilerParams(needs_layout_passes=False, use_tc_tiling_on_sc=False),
        scratch_types=[
            pltpu.VMEM((NCH, CH), jnp.int32),
            pltpu.VMEM((NCH, CH), jnp.int32),
            pltpu.VMEM((SUP,), jnp.float32),
            pltpu.VMEM((SUP, HD), jnp.float32),
            pltpu.VMEM((CH, HD), jnp.float32),
            pltpu.VMEM_SHARED((nd, HD), jnp.float32),
            pltpu.SemaphoreType.DMA,
            pltpu.SemaphoreType.DMA,
        ],
    )


BD = 1088


@functools.cache
def _build_k3(nr):
    nrt = nr // (NTILE * CH)
    prt = BD // 16
    rchunks = _chunks(prt, 32)

    def body(hp, gidc, outp, gidc_v, h_v, zb_v, out_sh):
        cid = lax.axis_index("c")
        sid = lax.axis_index("s")
        wid = cid * 16 + sid

        @plsc.parallel_loop(0, 32, 1)
        def _(r):
            for j in range(4):
                zb_v[r, pl.ds(j * NLANE, NLANE)] = jnp.zeros((NLANE,), jnp.float32)

        for off, rc in rchunks:
            pltpu.sync_copy(zb_v.at[pl.ds(0, rc), :],
                            out_sh.at[pl.ds(sid * prt + off, rc), :])
        plsc.subcore_barrier()

        pltpu.sync_copy(gidc.at[pl.ds(wid * nrt, nrt)], gidc_v)
        for c in range(nrt):
            pltpu.sync_copy(hp.at[pl.ds((wid * nrt + c) * CH, CH), :], h_v)
            pltpu.sync_copy(h_v, out_sh.at[gidc_v.at[c]], add=True)

        plsc.subcore_barrier()
        for off, rc in rchunks:
            sl = pl.ds(sid * prt + off, rc)
            pltpu.sync_copy(out_sh.at[sl, :], outp.at[cid].at[sl, :])

    return pl.kernel(
        body,
        out_type=jax.ShapeDtypeStruct((2, BD, HID), jnp.float32),
        mesh=_mesh(),
        compiler_params=pltpu.CompilerParams(needs_layout_passes=False, use_tc_tiling_on_sc=False),
        scratch_types=[
            pltpu.VMEM((nrt, CH), jnp.int32),
            pltpu.VMEM((CH, HID), jnp.float32),
            pltpu.VMEM((32, HID), jnp.float32),
            pltpu.VMEM_SHARED((BD, HID), jnp.float32),
        ],
    )


def _run_graph(x, eattr, ei, gids, W_node, W_edge, Wl, Wel, Asrc, Adst, Aedge, Woutl):
    n = x.shape[0]
    ne = ei.shape[1]
    nd = _cdiv(n + 1, SUP) * SUP
    nsup_tot = _cdiv(ne, NTILE * SUP) * NTILE
    ne_pad = nsup_tot * SUP
    nr = _cdiv(n, NTILE * CH) * NTILE * CH

    h = _mm(x, W_node)
    src = jnp.concatenate([ei[0], jnp.full((ne_pad - ne,), n, jnp.int32)])
    dst = jnp.concatenate([ei[1], jnp.full((ne_pad - ne,), n, jnp.int32)])
    srcc = src.reshape(nsup_tot, NCH, CH)
    dstc = dst.reshape(nsup_tot, NCH, CH)

    Cw = jnp.einsum('eh,lhkd,lkd->elk', W_edge, Wel.reshape(L, HID, HEADS, HD), Aedge)
    es_all = _mm(eattr, Cw.reshape(6, L * HEADS)).reshape(ne, L, HEADS)
    es_max = es_all.max(0)
    esh_T = jnp.transpose(es_all, (1, 2, 0))
    esh_T = jnp.pad(esh_T, ((0, 0), (0, 0), (0, ne_pad - ne)))
    esh_l = [esh_T[i].reshape(HEADS, nsup_tot, SUP) for i in range(L)]

    k1 = _build_k1(nd, nsup_tot)
    k2 = _build_k2(nd, nsup_tot)

    for i in range(L):
        Wh = _mm(h, Wl[i])
        Whr = Wh.reshape(n, HEADS, HD)
        ssrc = (Whr * Asrc[i]).sum(-1)
        sdst = (Whr * Adst[i]).sum(-1)
        Kh = ssrc.max(0) + es_max[i]

        tab = jnp.stack([ssrc - Kh[None, :], sdst + Kh[None, :]], axis=-1)
        tab = jnp.transpose(tab, (1, 0, 2)).reshape(HEADS, 2 * n)
        tabs = jnp.pad(tab, ((0, 0), (0, 2 * nd - 2 * n)))

        whh = jnp.transpose(Whr, (1, 0, 2))
        whh = jnp.pad(whh, ((0, 0), (0, nd - n), (0, 0)))

        eout, denp = k1(tabs, srcc, dstc, esh_l[i])
        invd = 1.0 / (denp[0] + denp[1] + 1e-16)

        aggp = k2(whh, srcc, dstc, eout)
        agg = (aggp[0] + aggp[1]) * invd[:, :, None]
        agg = jnp.transpose(agg[:, :n, :], (1, 0, 2)).reshape(n, HID)
        h = jax.nn.relu(h + _mm(agg, Woutl[i]))

    hp = jnp.pad(h, ((0, nr - n), (0, 0)))
    gid = jnp.concatenate([gids.astype(jnp.int32),
                           jnp.full((nr - n,), B, jnp.int32)])
    gidc = gid.reshape(nr // CH, CH)
    outp = _build_k3(nr)(hp, gidc)
    return (outp[0] + outp[1])[:B]


def kernel(x_main, edge_attr_main, edge_index_main, graph_ids_main,
           x_bb1, edge_attr_bb1, edge_index_bb1, graph_ids_bb1,
           x_bb2, edge_attr_bb2, edge_index_bb2, graph_ids_bb2,
           x_bb3, edge_attr_bb3, edge_index_bb3, graph_ids_bb3,
           protein_embedding, W_node, W_edge, W_prot, W_bb, W_final,
           Wl, Wel, Asrc, Adst, Aedge, Woutl):
    main_out = _run_graph(x_main, edge_attr_main, edge_index_main, graph_ids_main,
                          W_node, W_edge, Wl, Wel, Asrc, Adst, Aedge, Woutl)
    bb_outs = []
    for x, ea, ei, g in ((x_bb1, edge_attr_bb1, edge_index_bb1, graph_ids_bb1),
                         (x_bb2, edge_attr_bb2, edge_index_bb2, graph_ids_bb2),
                         (x_bb3, edge_attr_bb3, edge_index_bb3, graph_ids_bb3)):
        bb = _run_graph(x, ea, ei, g, W_node, W_edge, Wl, Wel, Asrc, Adst, Aedge, Woutl)
        bb_outs.append(_mm(bb, W_bb))
    prot = _mm(protein_embedding, W_prot)
    out = jnp.concatenate([main_out, prot] + bb_outs, axis=1) @ W_final
    return jax.nn.sigmoid(out)
